# baseline (speedup 1.0000x reference)
import jax
import jax.numpy as jnp
from jax.experimental import pallas as pl
from jax.experimental.pallas import tpu as pltpu

_NUM_EXPERTS = 16
_BLOCK_M = 2048

_DNT = (((1,), (1,)), ((), ()))
_DNN = (((1,), (0,)), ((), ()))


def _fused_gating_kernel(x_ref, w1_ref, b1_ref, w2_ref, b2_ref, w3_ref, b3_ref,
                         out_ref):
    x = x_ref[...]
    h = jnp.maximum(jax.lax.dot_general(w1_ref[...], x, _DNT) + b1_ref[...],
                    0.0)
    h = jnp.maximum(jax.lax.dot_general(w2_ref[...], h, _DNN) + b2_ref[...],
                    0.0)
    lt = jax.lax.dot_general(w3_ref[...], h, _DNN) + b3_ref[...]
    m = jnp.max(lt, axis=0, keepdims=True)
    row = jax.lax.broadcasted_iota(jnp.int32, lt.shape, 0)
    idx = jnp.min(jnp.where(lt == m, row, _NUM_EXPERTS), axis=0,
                  keepdims=True)
    onehot_t = (row == idx).astype(jnp.float32)
    eye = (jax.lax.broadcasted_iota(jnp.int32, (16, 16), 0)
           == jax.lax.broadcasted_iota(jnp.int32, (16, 16), 1)
           ).astype(jnp.float32)
    out_ref[...] = jax.lax.dot_general(onehot_t, eye,
                                       (((0,), (0,)), ((), ())))


def kernel(features, W1, b1, W2, b2, W3, b3):
    n_tokens, input_size = features.shape
    hidden = W1.shape[0]
    hidden2 = W2.shape[0]
    n_experts = W3.shape[0]

    b1c = b1.reshape(hidden, 1)
    b2c = b2.reshape(hidden2, 1)
    b3c = b3.reshape(n_experts, 1)

    bm = min(_BLOCK_M, n_tokens)
    grid = (n_tokens // bm,)

    onehot_t = pl.pallas_call(
        _fused_gating_kernel,
        grid=grid,
        in_specs=[
            pl.BlockSpec((bm, input_size), lambda i: (i, 0)),
            pl.BlockSpec((hidden, input_size), lambda i: (0, 0)),
            pl.BlockSpec((hidden, 1), lambda i: (0, 0)),
            pl.BlockSpec((hidden2, hidden), lambda i: (0, 0)),
            pl.BlockSpec((hidden2, 1), lambda i: (0, 0)),
            pl.BlockSpec((n_experts, hidden2), lambda i: (0, 0)),
            pl.BlockSpec((n_experts, 1), lambda i: (0, 0)),
        ],
        out_specs=pl.BlockSpec((bm, n_experts), lambda i: (i, 0)),
        out_shape=jax.ShapeDtypeStruct((n_tokens, n_experts), jnp.float32),
        compiler_params=pltpu.CompilerParams(
            dimension_semantics=("arbitrary",),
        ),
    )(features, W1, b1c, W2, b2c, W3, b3c)
    return onehot_t


# speedup vs baseline: 1.2101x; 1.2101x over previous
import jax
import jax.numpy as jnp
from jax.experimental import pallas as pl
from jax.experimental.pallas import tpu as pltpu

_NUM_EXPERTS = 16
_BLOCK_M = 2048

_DNT = (((1,), (1,)), ((), ()))
_DNN = (((1,), (0,)), ((), ()))


def _fused_gating_kernel(x_ref, w1_ref, b1_ref, w2_ref, b2_ref, w3_ref, b3_ref,
                         out_ref):
    x = x_ref[...]
    h = jnp.maximum(jax.lax.dot_general(w1_ref[...], x, _DNT) + b1_ref[...],
                    0.0)
    h = jnp.maximum(jax.lax.dot_general(w2_ref[...], h, _DNN) + b2_ref[...],
                    0.0)
    lt = jax.lax.dot_general(w3_ref[...], h, _DNN) + b3_ref[...]
    m = jnp.max(lt, axis=0, keepdims=True)
    row = jax.lax.broadcasted_iota(jnp.int32, lt.shape, 0)
    idx = jnp.min(jnp.where(lt == m, row, _NUM_EXPERTS), axis=0,
                  keepdims=True)
    out_ref[...] = (row == idx).astype(jnp.int8)


def kernel(features, W1, b1, W2, b2, W3, b3):
    n_tokens, input_size = features.shape
    hidden = W1.shape[0]
    hidden2 = W2.shape[0]
    n_experts = W3.shape[0]

    b1c = b1.reshape(hidden, 1)
    b2c = b2.reshape(hidden2, 1)
    b3c = b3.reshape(n_experts, 1)

    bm = min(_BLOCK_M, n_tokens)
    grid = (n_tokens // bm,)

    onehot_t = pl.pallas_call(
        _fused_gating_kernel,
        grid=grid,
        in_specs=[
            pl.BlockSpec((bm, input_size), lambda i: (i, 0)),
            pl.BlockSpec((hidden, input_size), lambda i: (0, 0)),
            pl.BlockSpec((hidden, 1), lambda i: (0, 0)),
            pl.BlockSpec((hidden2, hidden), lambda i: (0, 0)),
            pl.BlockSpec((hidden2, 1), lambda i: (0, 0)),
            pl.BlockSpec((n_experts, hidden2), lambda i: (0, 0)),
            pl.BlockSpec((n_experts, 1), lambda i: (0, 0)),
        ],
        out_specs=pl.BlockSpec((n_experts, bm), lambda i: (0, i)),
        out_shape=jax.ShapeDtypeStruct((n_experts, n_tokens), jnp.int8),
        compiler_params=pltpu.CompilerParams(
            dimension_semantics=("arbitrary",),
        ),
    )(features, W1, b1c, W2, b2c, W3, b3c)
    return onehot_t.T.astype(jnp.float32)


# speedup vs baseline: 1.2581x; 1.0397x over previous
import jax
import jax.numpy as jnp
from jax.experimental import pallas as pl
from jax.experimental.pallas import tpu as pltpu

_NUM_EXPERTS = 16
_BLOCK_M = 4096

_DNT = (((1,), (1,)), ((), ()))
_DNN = (((1,), (0,)), ((), ()))


def _fused_gating_kernel(x_ref, w1_ref, b1_ref, w2_ref, b2_ref, w3_ref, b3_ref,
                         out_ref):
    x = x_ref[...]
    h = jnp.maximum(jax.lax.dot_general(w1_ref[...], x, _DNT) + b1_ref[...],
                    0.0)
    h = jnp.maximum(jax.lax.dot_general(w2_ref[...], h, _DNN) + b2_ref[...],
                    0.0)
    lt = jax.lax.dot_general(w3_ref[...], h, _DNN) + b3_ref[...]
    m = jnp.max(lt, axis=0, keepdims=True)
    row = jax.lax.broadcasted_iota(jnp.int32, lt.shape, 0)
    idx = jnp.min(jnp.where(lt == m, row, _NUM_EXPERTS), axis=0,
                  keepdims=True)
    out_ref[...] = (row == idx).astype(jnp.float32)


def kernel(features, W1, b1, W2, b2, W3, b3):
    n_tokens, input_size = features.shape
    hidden = W1.shape[0]
    hidden2 = W2.shape[0]
    n_experts = W3.shape[0]

    b1c = b1.reshape(hidden, 1)
    b2c = b2.reshape(hidden2, 1)
    b3c = b3.reshape(n_experts, 1)

    bm = min(_BLOCK_M, n_tokens)
    grid = (n_tokens // bm,)

    onehot_t = pl.pallas_call(
        _fused_gating_kernel,
        grid=grid,
        in_specs=[
            pl.BlockSpec((bm, input_size), lambda i: (i, 0)),
            pl.BlockSpec((hidden, input_size), lambda i: (0, 0)),
            pl.BlockSpec((hidden, 1), lambda i: (0, 0)),
            pl.BlockSpec((hidden2, hidden), lambda i: (0, 0)),
            pl.BlockSpec((hidden2, 1), lambda i: (0, 0)),
            pl.BlockSpec((n_experts, hidden2), lambda i: (0, 0)),
            pl.BlockSpec((n_experts, 1), lambda i: (0, 0)),
        ],
        out_specs=pl.BlockSpec((n_experts, bm), lambda i: (0, i)),
        out_shape=jax.ShapeDtypeStruct((n_experts, n_tokens), jnp.float32),
        compiler_params=pltpu.CompilerParams(
            dimension_semantics=("arbitrary",),
        ),
    )(features, W1, b1c, W2, b2c, W3, b3c)
    return onehot_t.T


# speedup vs baseline: 1.2725x; 1.0115x over previous
import jax
import jax.numpy as jnp
from jax.experimental import pallas as pl
from jax.experimental.pallas import tpu as pltpu

_NUM_EXPERTS = 16
_BLOCK_M = 2048

_DNT = (((1,), (1,)), ((), ()))
_DNN = (((1,), (0,)), ((), ()))


def _fused_gating_kernel(x_ref, w1_ref, b1_ref, w2_ref, b2_ref, w3_ref, b3_ref,
                         out_ref):
    x = x_ref[...]
    h = jnp.maximum(jax.lax.dot_general(w1_ref[...], x, _DNT) + b1_ref[...],
                    0.0)
    h = jnp.maximum(jax.lax.dot_general(w2_ref[...], h, _DNN) + b2_ref[...],
                    0.0)
    lt = jax.lax.dot_general(w3_ref[...], h, _DNN) + b3_ref[...]
    m = jnp.max(lt, axis=0, keepdims=True)
    row = jax.lax.broadcasted_iota(jnp.int32, lt.shape, 0)
    idx = jnp.min(jnp.where(lt == m, row, _NUM_EXPERTS), axis=0,
                  keepdims=True)
    out_ref[...] = (row == idx).astype(jnp.float32)


def kernel(features, W1, b1, W2, b2, W3, b3):
    n_tokens, input_size = features.shape
    hidden = W1.shape[0]
    hidden2 = W2.shape[0]
    n_experts = W3.shape[0]

    b1c = b1.reshape(hidden, 1)
    b2c = b2.reshape(hidden2, 1)
    b3c = b3.reshape(n_experts, 1)

    bm = min(_BLOCK_M, n_tokens)
    grid = (n_tokens // bm,)

    onehot_t = pl.pallas_call(
        _fused_gating_kernel,
        grid=grid,
        in_specs=[
            pl.BlockSpec((bm, input_size), lambda i: (i, 0)),
            pl.BlockSpec((hidden, input_size), lambda i: (0, 0)),
            pl.BlockSpec((hidden, 1), lambda i: (0, 0)),
            pl.BlockSpec((hidden2, hidden), lambda i: (0, 0)),
            pl.BlockSpec((hidden2, 1), lambda i: (0, 0)),
            pl.BlockSpec((n_experts, hidden2), lambda i: (0, 0)),
            pl.BlockSpec((n_experts, 1), lambda i: (0, 0)),
        ],
        out_specs=pl.BlockSpec((n_experts, bm), lambda i: (0, i)),
        out_shape=jax.ShapeDtypeStruct((n_experts, n_tokens), jnp.float32),
        compiler_params=pltpu.CompilerParams(
            dimension_semantics=("arbitrary",),
        ),
    )(features, W1, b1c, W2, b2c, W3, b3c)
    return onehot_t.T


# speedup vs baseline: 1.3696x; 1.0763x over previous
import jax
import jax.numpy as jnp
from jax.experimental import pallas as pl
from jax.experimental.pallas import tpu as pltpu

_NUM_EXPERTS = 16
_BLOCK_M = 2048

_DNT = (((1,), (1,)), ((), ()))
_DNN = (((1,), (0,)), ((), ()))


def _fused_gating_kernel(x_ref, w1_ref, w2_ref, w3_ref, b_ref, out_ref):
    x = x_ref[...]
    b1 = b_ref[0:512, :]
    b2 = b_ref[512:768, :]
    b3 = b_ref[768:784, :]
    h = jnp.maximum(jax.lax.dot_general(w1_ref[...], x, _DNT) + b1, 0.0)
    h = jnp.maximum(jax.lax.dot_general(w2_ref[...], h, _DNN) + b2, 0.0)
    lt = jax.lax.dot_general(w3_ref[...], h, _DNN) + b3
    m = jnp.max(lt, axis=0, keepdims=True)
    row = jax.lax.broadcasted_iota(jnp.int32, lt.shape, 0)
    idx = jnp.min(jnp.where(lt == m, row, _NUM_EXPERTS), axis=0,
                  keepdims=True)
    out_ref[...] = (row == idx).astype(jnp.float32)


def kernel(features, W1, b1, W2, b2, W3, b3):
    n_tokens, input_size = features.shape
    hidden = W1.shape[0]
    hidden2 = W2.shape[0]
    n_experts = W3.shape[0]

    bcol = jnp.concatenate([b1, b2, b3]).reshape(-1, 1)
    n_b = hidden + hidden2 + n_experts

    bm = min(_BLOCK_M, n_tokens)
    grid = (n_tokens // bm,)

    onehot_t = pl.pallas_call(
        _fused_gating_kernel,
        grid=grid,
        in_specs=[
            pl.BlockSpec((bm, input_size), lambda i: (i, 0)),
            pl.BlockSpec((hidden, input_size), lambda i: (0, 0)),
            pl.BlockSpec((hidden2, hidden), lambda i: (0, 0)),
            pl.BlockSpec((n_experts, hidden2), lambda i: (0, 0)),
            pl.BlockSpec((n_b, 1), lambda i: (0, 0)),
        ],
        out_specs=pl.BlockSpec((n_experts, bm), lambda i: (0, i)),
        out_shape=jax.ShapeDtypeStruct((n_experts, n_tokens), jnp.float32),
        compiler_params=pltpu.CompilerParams(
            dimension_semantics=("arbitrary",),
        ),
    )(features, W1, W2, W3, bcol)
    return onehot_t.T


# speedup vs baseline: 1.3714x; 1.0013x over previous
import jax
import jax.numpy as jnp
from jax.experimental import pallas as pl
from jax.experimental.pallas import tpu as pltpu

_NUM_EXPERTS = 16
_BLOCK_M = 2048

_DNT = (((1,), (1,)), ((), ()))
_DNN = (((1,), (0,)), ((), ()))


def _fused_gating_kernel(x_ref, w1_ref, w2_ref, w3_ref, b_ref, out_ref):
    x = x_ref[...]
    b1 = b_ref[0:512, :]
    b2 = b_ref[512:768, :]
    b3 = b_ref[768:784, :]
    h = jnp.maximum(jax.lax.dot_general(w1_ref[...], x, _DNT) + b1, 0.0)
    h = jnp.maximum(jax.lax.dot_general(w2_ref[...], h, _DNN) + b2, 0.0)
    lt = jax.lax.dot_general(w3_ref[...], h, _DNN) + b3
    m = jnp.max(lt, axis=0, keepdims=True)
    row = jax.lax.broadcasted_iota(jnp.int32, lt.shape, 0)
    idx = jnp.min(jnp.where(lt == m, row, _NUM_EXPERTS), axis=0,
                  keepdims=True)
    out_ref[...] = (row == idx).astype(jnp.float32)


def kernel(features, W1, b1, W2, b2, W3, b3):
    n_tokens, input_size = features.shape
    hidden = W1.shape[0]
    hidden2 = W2.shape[0]
    n_experts = W3.shape[0]

    bcol = jnp.concatenate([b1, b2, b3]).reshape(-1, 1)
    n_b = hidden + hidden2 + n_experts

    bm = min(_BLOCK_M, n_tokens)
    grid = (n_tokens // bm,)

    onehot_t = pl.pallas_call(
        _fused_gating_kernel,
        grid=grid,
        in_specs=[
            pl.BlockSpec((bm, input_size), lambda i: (i, 0)),
            pl.BlockSpec((hidden, input_size), lambda i: (0, 0)),
            pl.BlockSpec((hidden2, hidden), lambda i: (0, 0)),
            pl.BlockSpec((n_experts, hidden2), lambda i: (0, 0)),
            pl.BlockSpec((n_b, 1), lambda i: (0, 0)),
        ],
        out_specs=pl.BlockSpec((n_experts, bm), lambda i: (0, i)),
        out_shape=jax.ShapeDtypeStruct((n_experts, n_tokens), jnp.float32),
        compiler_params=pltpu.CompilerParams(
            dimension_semantics=("parallel",),
        ),
    )(features, W1, W2, W3, bcol)
    return onehot_t.T
